# baseline (speedup 1.0000x reference)
import functools

import jax
import jax.numpy as jnp
import numpy as np
from jax import lax
from jax.experimental import pallas as pl
from jax.experimental.pallas import tpu as pltpu
from jax.experimental.pallas import tpu_sc as plsc

N = 10000
E = 320000
D_IN = 128
H = 64
L_LAYERS = 6
G = 64

NC = 2
NS = 16
NW = NC * NS
CHUNK = 128
NCHUNK = 80
EPT = NCHUNK * CHUNK
E_PAD = NW * EPT
N_PAD = 10240
RPT = N_PAD // NS
DEG_W = 16

_PERM = np.concatenate([
    np.stack([np.arange(32 * g, 32 * g + 16),
              np.arange(32 * g + 16, 32 * g + 32)], axis=1).reshape(-1)
    for g in range(H // 32)
])

_mesh = plsc.VectorSubcoreMesh(
    core_axis_name="c", subcore_axis_name="s", num_cores=NC, num_subcores=NS)

_f32 = jnp.float32


@functools.partial(
    pl.kernel,
    out_type=jax.ShapeDtypeStruct((NC, N_PAD, DEG_W), _f32),
    mesh=_mesh,
    scratch_types=[
        pltpu.VMEM((NCHUNK, CHUNK), jnp.int32),
        pltpu.VMEM((CHUNK, DEG_W), _f32),
        pltpu.VMEM_SHARED((N_PAD, DEG_W), _f32),
    ],
    compiler_params=pltpu.CompilerParams(use_tc_tiling_on_sc=False),
)
def _deg_sc(dst_hbm, out_hbm, idx_v, ones_v, dsh):
    c = lax.axis_index("c")
    s = lax.axis_index("s")
    wid = s * NC + c

    def fill(i, carry):
        ones_v[i, :] = jnp.zeros((DEG_W,), _f32)
        return carry
    lax.fori_loop(0, CHUNK, fill, 0)
    base = s * RPT
    for t in range(RPT // CHUNK):
        pltpu.sync_copy(ones_v, dsh.at[pl.ds(base + t * CHUNK, CHUNK), :])

    def fill1(i, carry):
        ones_v[i, :] = jnp.full((DEG_W,), 1.0, _f32)
        return carry
    lax.fori_loop(0, CHUNK, fill1, 0)

    plsc.subcore_barrier()
    pltpu.sync_copy(dst_hbm.at[wid], idx_v)

    def body(j, carry):
        pltpu.sync_copy(ones_v, dsh.at[idx_v.at[j]], add=True)
        return carry
    lax.fori_loop(0, NCHUNK, body, 0)

    plsc.subcore_barrier()
    pltpu.sync_copy(dsh.at[pl.ds(base, RPT), :],
                    out_hbm.at[c, pl.ds(base, RPT), :])


@functools.partial(
    pl.kernel,
    out_type=jax.ShapeDtypeStruct((NC, N_PAD, H), _f32),
    mesh=_mesh,
    scratch_types=[
        pltpu.VMEM((NCHUNK, CHUNK), jnp.int32),
        pltpu.VMEM((NCHUNK, CHUNK), jnp.int32),
        pltpu.VMEM((4, CHUNK, H), jnp.bfloat16),
        pltpu.VMEM((4, CHUNK, H), jnp.bfloat16),
        pltpu.VMEM((4, CHUNK, H), _f32),
        pltpu.VMEM_SHARED((N_PAD, H), _f32),
        [pltpu.SemaphoreType.DMA] * 4,
        [pltpu.SemaphoreType.DMA] * 4,
        [pltpu.SemaphoreType.DMA] * 4,
    ],
    compiler_params=pltpu.CompilerParams(use_tc_tiling_on_sc=False,
                                         needs_layout_passes=False),
)
def _edge_sc(p_hbm, q_hbm, src_hbm, dst_hbm, out_hbm,
             src_v, dst_v, pbuf, qbuf, rbuf, ssh, sem_p, sem_q, sem_s):
    c = lax.axis_index("c")
    s = lax.axis_index("s")
    wid = s * NC + c

    zb = rbuf.at[0]

    def fill(i, carry):
        for k in range(H // 16):
            zb[i, pl.ds(k * 16, 16)] = jnp.zeros((16,), _f32)
        return carry
    lax.fori_loop(0, CHUNK, fill, 0)
    base = s * RPT
    for t in range(RPT // CHUNK):
        pltpu.sync_copy(zb, ssh.at[pl.ds(base + t * CHUNK, CHUNK), :])
    plsc.subcore_barrier()

    pltpu.sync_copy(src_hbm.at[wid], src_v)
    pltpu.sync_copy(dst_hbm.at[wid], dst_v)

    def issue(jj, b):
        pltpu.async_copy(p_hbm.at[dst_v.at[jj]], pbuf.at[b], sem_p[b])
        pltpu.async_copy(q_hbm.at[src_v.at[jj]], qbuf.at[b], sem_q[b])

    def consume(jj, b):
        pb = pbuf.at[b]
        qb = qbuf.at[b]
        rb = rbuf.at[b]
        dj = dst_v.at[jj]
        pltpu.make_async_copy(p_hbm.at[dj], pb, sem_p[b]).wait()
        pltpu.make_async_copy(q_hbm.at[src_v.at[jj]], qb, sem_q[b]).wait()
        zero32 = jnp.zeros((32,), jnp.bfloat16)

        def row(i, carry2):
            for u in range(4):
                r = 4 * i + u
                for g in range(H // 32):
                    sl = pl.ds(32 * g, 32)
                    y = jnp.maximum(pb[r, sl] + qb[r, sl], zero32)
                    lo, hi = plsc.unpack(y, format=plsc.PackFormat.INTERLEAVED)
                    rb[r, pl.ds(32 * g, 16)] = lo
                    rb[r, pl.ds(32 * g + 16, 16)] = hi
            return carry2
        lax.fori_loop(0, CHUNK // 4, row, 0)
        pltpu.async_copy(rb, ssh.at[dj], sem_s[b], add=True)

    def drain(jj, b):
        pltpu.make_async_copy(rbuf.at[b], ssh.at[dst_v.at[jj]], sem_s[b]).wait()

    issue(0, 0)
    issue(1, 1)
    consume(0, 0)
    issue(2, 2)
    consume(1, 1)
    issue(3, 3)

    def body(j, carry):
        m0 = 4 * j + 2
        for u in range(4):
            m = m0 + u
            b = (2 + u) % 4
            consume(m, b)
            drain(m - 2, (b + 2) % 4)
            issue(m + 2, (b + 2) % 4)
        return carry
    lax.fori_loop(0, (NCHUNK - 4) // 4, body, 0)
    consume(NCHUNK - 2, 2)
    drain(NCHUNK - 4, 0)
    consume(NCHUNK - 1, 3)
    drain(NCHUNK - 3, 1)
    drain(NCHUNK - 2, 2)
    drain(NCHUNK - 1, 3)

    plsc.subcore_barrier()
    pltpu.sync_copy(ssh.at[pl.ds(base, RPT), :],
                    out_hbm.at[c, pl.ds(base, RPT), :])


def _mm(a, b):
    return jnp.dot(a, b, preferred_element_type=_f32)


def _pq_from_h(h, cw1_ref, cb1_ref, p_ref, q_ref):
    w1a = cw1_ref[0:H, :]
    w1b = cw1_ref[H:2 * H, :]
    bf = jnp.bfloat16
    p_ref[0:N, :] = (_mm(h, w1a - w1b) + cb1_ref[...]).astype(bf)
    p_ref[N:N_PAD, :] = jnp.zeros((N_PAD - N, H), bf)
    q_ref[0:N, :] = _mm(h, w1b).astype(bf)
    q_ref[N:N_PAD, :] = jnp.zeros((N_PAD - N, H), bf)


def _encode_pq_body(x_ref, ew1, eb1, ew2, eb2, cw1, cb1, h_ref, p_ref, q_ref):
    z = jnp.maximum(_mm(x_ref[...], ew1[...]) + eb1[...], 0.0)
    h = _mm(z, ew2[...]) + eb2[...]
    h_ref[...] = h
    _pq_from_h(h, cw1, cb1, p_ref, q_ref)


def _agg_h(h_ref, s2_ref, d2_ref, w2, b2):
    s = s2_ref[0, 0:N, :] + s2_ref[1, 0:N, :]
    dsum = d2_ref[0, 0:N, :] + d2_ref[1, 0:N, :]
    dcol = dsum[:, 0:1]
    inv = 1.0 / jnp.maximum(dcol, 1.0)
    has_edge = jnp.minimum(dcol, 1.0)
    agg = _mm(s * inv, w2[...]) + b2[...] * has_edge
    return h_ref[...] + agg


def _update_pq_body(h_ref, s2_ref, d2_ref, w2, b2, cw1n, cb1n,
                    ho_ref, p_ref, q_ref):
    hn = _agg_h(h_ref, s2_ref, d2_ref, w2, b2)
    ho_ref[...] = hn
    _pq_from_h(hn, cw1n, cb1n, p_ref, q_ref)


def _update_head_body(h_ref, s2_ref, d2_ref, w2, b2,
                      hw1, hb1, hw2, hb2, pred_ref):
    hn = _agg_h(h_ref, s2_ref, d2_ref, w2, b2)
    z = jnp.maximum(_mm(hn, hw1[...]) + hb1[...], 0.0)
    pred_ref[...] = _mm(z, hw2[...]) + hb2[...]


def _segmax_body(pred_ref, batch_ref, out_ref):
    pred2 = pred_ref[...]
    b2d = batch_ref[...]
    neg = jnp.full((80, 125), -jnp.inf, _f32)
    gids = lax.broadcasted_iota(jnp.int32, (G, 1), 0)

    def seg(g, acc):
        m = jnp.max(jnp.where(b2d == g, pred2, neg), axis=0, keepdims=True)
        return jnp.where(gids == g, m, acc)
    acc = lax.fori_loop(0, G, seg, jnp.full((G, 125), -jnp.inf, _f32))
    out_ref[...] = jnp.max(acc, axis=1, keepdims=True)


_encode_pq = pl.pallas_call(
    _encode_pq_body,
    out_shape=(jax.ShapeDtypeStruct((N, H), _f32),
               jax.ShapeDtypeStruct((N_PAD, H), jnp.bfloat16),
               jax.ShapeDtypeStruct((N_PAD, H), jnp.bfloat16)))

_update_pq = pl.pallas_call(
    _update_pq_body,
    out_shape=(jax.ShapeDtypeStruct((N, H), _f32),
               jax.ShapeDtypeStruct((N_PAD, H), jnp.bfloat16),
               jax.ShapeDtypeStruct((N_PAD, H), jnp.bfloat16)))

_update_head = pl.pallas_call(
    _update_head_body,
    out_shape=jax.ShapeDtypeStruct((N, 1), _f32))

_segmax = pl.pallas_call(
    _segmax_body,
    out_shape=jax.ShapeDtypeStruct((G, 1), _f32))


def kernel(x, edge_index, batch,
           enc_W1, enc_b1, enc_W2, enc_b2,
           conv_W1, conv_b1, conv_W2, conv_b2,
           head_W1, head_b1, head_W2, head_b2):
    src = edge_index[0].astype(jnp.int32)
    dst = edge_index[1].astype(jnp.int32)
    pad_idx = N + (jnp.arange(E_PAD - E, dtype=jnp.int32) % (N_PAD - N))
    src_p = jnp.concatenate([src, pad_idx]).reshape(NW, NCHUNK, CHUNK)
    dst_p = jnp.concatenate([dst, pad_idx]).reshape(NW, NCHUNK, CHUNK)

    d2 = _deg_sc(dst_p)

    cw1p = conv_W1[:, :, _PERM]
    cb1p = conv_b1[:, _PERM]

    eb1 = enc_b1.reshape(1, H)
    eb2 = enc_b2.reshape(1, H)
    h, p, q = _encode_pq(x, enc_W1, eb1, enc_W2, eb2,
                         cw1p[0], cb1p[0].reshape(1, H))

    out = None
    for l in range(L_LAYERS):
        s2 = _edge_sc(p, q, src_p, dst_p)
        w2 = conv_W2[l]
        b2 = conv_b2[l].reshape(1, H)
        if l + 1 < L_LAYERS:
            h, p, q = _update_pq(h, s2, d2, w2, b2,
                                 cw1p[l + 1], cb1p[l + 1].reshape(1, H))
        else:
            pred = _update_head(h, s2, d2, w2, b2,
                                head_W1, head_b1.reshape(1, H),
                                head_W2, head_b2.reshape(1, 1))
            out = _segmax(pred.reshape(80, 125),
                          batch.astype(jnp.int32).reshape(80, 125))
    return out


# speedup vs baseline: 1.5158x; 1.5158x over previous
import functools

import jax
import jax.numpy as jnp
from jax import lax
from jax.experimental import pallas as pl
from jax.experimental.pallas import tpu as pltpu
from jax.experimental.pallas import tpu_sc as plsc

N = 10000
E = 320000
D_IN = 128
H = 64
L_LAYERS = 6
G = 64

NC = 2
NS = 16
NW = NC * NS
CHUNK = 128
NCHUNK = 80
EPT = NCHUNK * CHUNK
E_PAD = NW * EPT
N_PAD = 10240
RPT = N_PAD // NS
DEG_W = 16

_mesh = plsc.VectorSubcoreMesh(
    core_axis_name="c", subcore_axis_name="s", num_cores=NC, num_subcores=NS)

_f32 = jnp.float32


@functools.partial(
    pl.kernel,
    out_type=jax.ShapeDtypeStruct((NC, N_PAD, DEG_W), _f32),
    mesh=_mesh,
    scratch_types=[
        pltpu.VMEM((NCHUNK, CHUNK), jnp.int32),
        pltpu.VMEM((CHUNK, DEG_W), _f32),
        pltpu.VMEM_SHARED((N_PAD, DEG_W), _f32),
    ],
    compiler_params=pltpu.CompilerParams(use_tc_tiling_on_sc=False),
)
def _deg_sc(dst_hbm, out_hbm, idx_v, ones_v, dsh):
    c = lax.axis_index("c")
    s = lax.axis_index("s")
    wid = s * NC + c

    def fill(i, carry):
        ones_v[i, :] = jnp.zeros((DEG_W,), _f32)
        return carry
    lax.fori_loop(0, CHUNK, fill, 0)
    base = s * RPT
    for t in range(RPT // CHUNK):
        pltpu.sync_copy(ones_v, dsh.at[pl.ds(base + t * CHUNK, CHUNK), :])

    def fill1(i, carry):
        ones_v[i, :] = jnp.full((DEG_W,), 1.0, _f32)
        return carry
    lax.fori_loop(0, CHUNK, fill1, 0)

    plsc.subcore_barrier()
    pltpu.sync_copy(dst_hbm.at[wid], idx_v)

    def body(j, carry):
        pltpu.sync_copy(ones_v, dsh.at[idx_v.at[j]], add=True)
        return carry
    lax.fori_loop(0, NCHUNK, body, 0)

    plsc.subcore_barrier()
    pltpu.sync_copy(dsh.at[pl.ds(base, RPT), :],
                    out_hbm.at[c, pl.ds(base, RPT), :])


@functools.partial(
    pl.kernel,
    out_type=jax.ShapeDtypeStruct((NC, N_PAD, H), _f32),
    mesh=_mesh,
    scratch_types=[
        pltpu.VMEM((NCHUNK, CHUNK), jnp.int32),
        pltpu.VMEM((NCHUNK, CHUNK), jnp.int32),
        pltpu.VMEM((4, CHUNK, H), _f32),
        pltpu.VMEM((4, CHUNK, H), _f32),
        pltpu.VMEM_SHARED((N_PAD, H), _f32),
        [pltpu.SemaphoreType.DMA] * 4,
        [pltpu.SemaphoreType.DMA] * 4,
        [pltpu.SemaphoreType.DMA] * 4,
    ],
    compiler_params=pltpu.CompilerParams(use_tc_tiling_on_sc=False),
)
def _edge_sc(p_hbm, q_hbm, src_hbm, dst_hbm, out_hbm,
             src_v, dst_v, pbuf, qbuf, ssh, sem_p, sem_q, sem_s):
    c = lax.axis_index("c")
    s = lax.axis_index("s")
    wid = s * NC + c

    zb = pbuf.at[0]

    def fill(i, carry):
        for k in range(H // 16):
            zb[i, pl.ds(k * 16, 16)] = jnp.zeros((16,), _f32)
        return carry
    lax.fori_loop(0, CHUNK, fill, 0)
    base = s * RPT
    for t in range(RPT // CHUNK):
        pltpu.sync_copy(zb, ssh.at[pl.ds(base + t * CHUNK, CHUNK), :])
    plsc.subcore_barrier()

    pltpu.sync_copy(src_hbm.at[wid], src_v)
    pltpu.sync_copy(dst_hbm.at[wid], dst_v)

    def issue(jj, b):
        pltpu.async_copy(p_hbm.at[dst_v.at[jj]], pbuf.at[b], sem_p[b])
        pltpu.async_copy(q_hbm.at[src_v.at[jj]], qbuf.at[b], sem_q[b])

    def consume(jj, b):
        pb = pbuf.at[b]
        qb = qbuf.at[b]
        dj = dst_v.at[jj]
        pltpu.make_async_copy(p_hbm.at[dj], pb, sem_p[b]).wait()
        pltpu.make_async_copy(q_hbm.at[src_v.at[jj]], qb, sem_q[b]).wait()

        def row(i, carry2):
            for u in range(4):
                r = 4 * i + u
                for k in range(H // 16):
                    sl = pl.ds(k * 16, 16)
                    pb[r, sl] = jnp.maximum(pb[r, sl] + qb[r, sl], 0.0)
            return carry2
        lax.fori_loop(0, CHUNK // 4, row, 0)
        pltpu.async_copy(pb, ssh.at[dj], sem_s[b], add=True)

    def drain(jj, b):
        pltpu.make_async_copy(pbuf.at[b], ssh.at[dst_v.at[jj]], sem_s[b]).wait()

    issue(0, 0)
    issue(1, 1)
    issue(2, 2)
    consume(0, 0)
    issue(3, 3)

    def body(j, carry):
        m0 = 4 * j + 1
        for u in range(4):
            m = m0 + u
            b = (1 + u) % 4
            consume(m, b)
            drain(m - 1, (b + 3) % 4)
            issue(m + 3, (b + 3) % 4)
        return carry
    lax.fori_loop(0, (NCHUNK - 4) // 4, body, 0)
    consume(NCHUNK - 3, 1)
    consume(NCHUNK - 2, 2)
    consume(NCHUNK - 1, 3)
    drain(NCHUNK - 4, 0)
    drain(NCHUNK - 3, 1)
    drain(NCHUNK - 2, 2)
    drain(NCHUNK - 1, 3)

    plsc.subcore_barrier()
    pltpu.sync_copy(ssh.at[pl.ds(base, RPT), :],
                    out_hbm.at[c, pl.ds(base, RPT), :])


def _mm(a, b):
    return jnp.dot(a, b, preferred_element_type=_f32)


def _pq_from_h(h, cw1_ref, cb1_ref, p_ref, q_ref):
    w1a = cw1_ref[0:H, :]
    w1b = cw1_ref[H:2 * H, :]
    p_ref[0:N, :] = _mm(h, w1a - w1b) + cb1_ref[...]
    p_ref[N:N_PAD, :] = jnp.zeros((N_PAD - N, H), _f32)
    q_ref[0:N, :] = _mm(h, w1b)
    q_ref[N:N_PAD, :] = jnp.zeros((N_PAD - N, H), _f32)


def _encode_pq_body(x_ref, ew1, eb1, ew2, eb2, cw1, cb1, h_ref, p_ref, q_ref):
    z = jnp.maximum(_mm(x_ref[...], ew1[...]) + eb1[...], 0.0)
    h = _mm(z, ew2[...]) + eb2[...]
    h_ref[...] = h
    _pq_from_h(h, cw1, cb1, p_ref, q_ref)


def _agg_h(h_ref, s2_ref, d2_ref, w2, b2):
    s = s2_ref[0, 0:N, :] + s2_ref[1, 0:N, :]
    dsum = d2_ref[0, 0:N, :] + d2_ref[1, 0:N, :]
    dcol = dsum[:, 0:1]
    inv = 1.0 / jnp.maximum(dcol, 1.0)
    has_edge = jnp.minimum(dcol, 1.0)
    agg = _mm(s * inv, w2[...]) + b2[...] * has_edge
    return h_ref[...] + agg


def _update_pq_body(h_ref, s2_ref, d2_ref, w2, b2, cw1n, cb1n,
                    ho_ref, p_ref, q_ref):
    hn = _agg_h(h_ref, s2_ref, d2_ref, w2, b2)
    ho_ref[...] = hn
    _pq_from_h(hn, cw1n, cb1n, p_ref, q_ref)


def _update_head_body(h_ref, s2_ref, d2_ref, w2, b2,
                      hw1, hb1, hw2, hb2, pred_ref):
    hn = _agg_h(h_ref, s2_ref, d2_ref, w2, b2)
    z = jnp.maximum(_mm(hn, hw1[...]) + hb1[...], 0.0)
    pred_ref[...] = _mm(z, hw2[...]) + hb2[...]


def _segmax_body(pred_ref, batch_ref, out_ref):
    pred2 = pred_ref[...]
    b2d = batch_ref[...]
    neg = jnp.full((80, 125), -jnp.inf, _f32)
    gids = lax.broadcasted_iota(jnp.int32, (G, 1), 0)

    def seg(g, acc):
        m = jnp.max(jnp.where(b2d == g, pred2, neg), axis=0, keepdims=True)
        return jnp.where(gids == g, m, acc)
    acc = lax.fori_loop(0, G, seg, jnp.full((G, 125), -jnp.inf, _f32))
    out_ref[...] = jnp.max(acc, axis=1, keepdims=True)


_encode_pq = pl.pallas_call(
    _encode_pq_body,
    out_shape=(jax.ShapeDtypeStruct((N, H), _f32),
               jax.ShapeDtypeStruct((N_PAD, H), _f32),
               jax.ShapeDtypeStruct((N_PAD, H), _f32)))

_update_pq = pl.pallas_call(
    _update_pq_body,
    out_shape=(jax.ShapeDtypeStruct((N, H), _f32),
               jax.ShapeDtypeStruct((N_PAD, H), _f32),
               jax.ShapeDtypeStruct((N_PAD, H), _f32)))

_update_head = pl.pallas_call(
    _update_head_body,
    out_shape=jax.ShapeDtypeStruct((N, 1), _f32))

_segmax = pl.pallas_call(
    _segmax_body,
    out_shape=jax.ShapeDtypeStruct((G, 1), _f32))


def kernel(x, edge_index, batch,
           enc_W1, enc_b1, enc_W2, enc_b2,
           conv_W1, conv_b1, conv_W2, conv_b2,
           head_W1, head_b1, head_W2, head_b2):
    src = edge_index[0].astype(jnp.int32)
    dst = edge_index[1].astype(jnp.int32)
    pad_idx = N + (jnp.arange(E_PAD - E, dtype=jnp.int32) % (N_PAD - N))
    src_p = jnp.concatenate([src, pad_idx]).reshape(NW, NCHUNK, CHUNK)
    dst_p = jnp.concatenate([dst, pad_idx]).reshape(NW, NCHUNK, CHUNK)

    d2 = _deg_sc(dst_p)

    eb1 = enc_b1.reshape(1, H)
    eb2 = enc_b2.reshape(1, H)
    h, p, q = _encode_pq(x, enc_W1, eb1, enc_W2, eb2,
                         conv_W1[0], conv_b1[0].reshape(1, H))

    out = None
    for l in range(L_LAYERS):
        s2 = _edge_sc(p, q, src_p, dst_p)
        w2 = conv_W2[l]
        b2 = conv_b2[l].reshape(1, H)
        if l + 1 < L_LAYERS:
            h, p, q = _update_pq(h, s2, d2, w2, b2,
                                 conv_W1[l + 1], conv_b1[l + 1].reshape(1, H))
        else:
            pred = _update_head(h, s2, d2, w2, b2,
                                head_W1, head_b1.reshape(1, H),
                                head_W2, head_b2.reshape(1, 1))
            out = _segmax(pred.reshape(80, 125),
                          batch.astype(jnp.int32).reshape(80, 125))
    return out
